# baseline (device time: 16499 ns/iter reference)
import jax
import jax.numpy as jnp
from jax import lax
from jax.experimental import pallas as pl
from jax.experimental.pallas import tpu as pltpu

N_DEV = 4
M = 1024
K_SHARD = 512
N = 1024
CHUNK = M // N_DEV
HALF = CHUNK // 2

QCLIP = 120.0


def kernel(A, B):
    def body(a_ref, b_ref, out_ref, a_bf, b_bf, pbuf, rbuf, send_sems, recv_sems):
        my = lax.axis_index("i")

        barrier_sem = pltpu.get_barrier_semaphore()
        for j in range(1, N_DEV):
            pl.semaphore_signal(
                barrier_sem,
                inc=1,
                device_id=((my + j) % N_DEV,),
                device_id_type=pl.DeviceIdType.MESH,
            )
        b_bf[:, :] = b_ref[:, :].astype(jnp.bfloat16)
        a_bf[:, :] = a_ref[:, :].astype(jnp.bfloat16)
        pl.semaphore_wait(barrier_sem, N_DEV - 1)

        def half_partial(c, h):
            return jnp.dot(
                a_bf[pl.ds(c * CHUNK + h * HALF, HALF), :],
                b_bf[:, :],
                preferred_element_type=jnp.float32,
            )

        def quant(p):
            return jnp.rint(
                jnp.clip(p, -QCLIP, QCLIP) * (127.0 / QCLIP)
            ).astype(jnp.int8)

        sends = []
        for j in range(1, N_DEV):
            target = (my + j) % N_DEV
            slot = N_DEV - j
            for h in range(2):
                pbuf[j, h] = quant(half_partial(target, h))
                rdma = pltpu.make_async_remote_copy(
                    src_ref=pbuf.at[j, h],
                    dst_ref=rbuf.at[slot, h],
                    send_sem=send_sems.at[j, h],
                    recv_sem=recv_sems.at[slot, h],
                    device_id=(target,),
                    device_id_type=pl.DeviceIdType.MESH,
                )
                rdma.start()
                sends.append(rdma)

        for h in range(2):
            acc = half_partial(my, h)
            for k in range(1, N_DEV):
                recv = pltpu.make_async_remote_copy(
                    src_ref=pbuf.at[k, h],
                    dst_ref=rbuf.at[k, h],
                    send_sem=send_sems.at[k, h],
                    recv_sem=recv_sems.at[k, h],
                    device_id=((my + k) % N_DEV,),
                    device_id_type=pl.DeviceIdType.MESH,
                )
                recv.wait_recv()
                acc = acc + rbuf[k, h].astype(jnp.float32) * (QCLIP / 127.0)
            out_ref[pl.ds(h * HALF, HALF), :] = acc

        for rdma in sends:
            rdma.wait_send()

    return pl.pallas_call(
        body,
        out_shape=jax.ShapeDtypeStruct((CHUNK, N), jnp.float32),
        in_specs=[
            pl.BlockSpec(memory_space=pltpu.VMEM),
            pl.BlockSpec(memory_space=pltpu.VMEM),
        ],
        out_specs=pl.BlockSpec(memory_space=pltpu.VMEM),
        scratch_shapes=[
            pltpu.VMEM((M, K_SHARD), jnp.bfloat16),
            pltpu.VMEM((K_SHARD, N), jnp.bfloat16),
            pltpu.VMEM((N_DEV, 2, HALF, N), jnp.int8),
            pltpu.VMEM((N_DEV, 2, HALF, N), jnp.int8),
            pltpu.SemaphoreType.DMA((N_DEV, 2)),
            pltpu.SemaphoreType.DMA((N_DEV, 2)),
        ],
        compiler_params=pltpu.CompilerParams(collective_id=0),
    )(A, B)


# device time: 16213 ns/iter; 1.0176x vs baseline; 1.0176x over previous
import jax
import jax.numpy as jnp
from jax import lax
from jax.experimental import pallas as pl
from jax.experimental.pallas import tpu as pltpu

N_DEV = 4
M = 1024
K_SHARD = 512
N = 1024
CHUNK = M // N_DEV
HALF = CHUNK // 2

QCLIP = 120.0


def kernel(A, B):
    def body(a_ref, b_ref, out_ref, a_bf, b_bf, pbuf, rbuf, send_sems, recv_sems):
        my = lax.axis_index("i")

        barrier_sem = pltpu.get_barrier_semaphore()
        for j in range(1, N_DEV):
            pl.semaphore_signal(
                barrier_sem,
                inc=1,
                device_id=((my + j) % N_DEV,),
                device_id_type=pl.DeviceIdType.MESH,
            )
        b_bf[:, :] = b_ref[:, :].astype(jnp.bfloat16)
        a_bf[:, :] = a_ref[:, :].astype(jnp.bfloat16)

        def half_partial(c, h):
            return jnp.dot(
                a_bf[pl.ds(c * CHUNK + h * HALF, HALF), :],
                b_bf[:, :],
                preferred_element_type=jnp.float32,
            )

        def quant(p):
            return jnp.rint(
                jnp.clip(p, -QCLIP, QCLIP) * (127.0 / QCLIP)
            ).astype(jnp.int8)

        pbuf[1, 0] = quant(half_partial((my + 1) % N_DEV, 0))
        pl.semaphore_wait(barrier_sem, N_DEV - 1)

        sends = []
        for j in range(1, N_DEV):
            target = (my + j) % N_DEV
            slot = N_DEV - j
            for h in range(2):
                if (j, h) != (1, 0):
                    pbuf[j, h] = quant(half_partial(target, h))
                rdma = pltpu.make_async_remote_copy(
                    src_ref=pbuf.at[j, h],
                    dst_ref=rbuf.at[slot, h],
                    send_sem=send_sems.at[j, h],
                    recv_sem=recv_sems.at[slot, h],
                    device_id=(target,),
                    device_id_type=pl.DeviceIdType.MESH,
                )
                rdma.start()
                sends.append(rdma)

        for h in range(2):
            acc = half_partial(my, h)
            for k in range(1, N_DEV):
                recv = pltpu.make_async_remote_copy(
                    src_ref=pbuf.at[k, h],
                    dst_ref=rbuf.at[k, h],
                    send_sem=send_sems.at[k, h],
                    recv_sem=recv_sems.at[k, h],
                    device_id=((my + k) % N_DEV,),
                    device_id_type=pl.DeviceIdType.MESH,
                )
                recv.wait_recv()
                acc = acc + rbuf[k, h].astype(jnp.float32) * (QCLIP / 127.0)
            out_ref[pl.ds(h * HALF, HALF), :] = acc

        for rdma in sends:
            rdma.wait_send()

    return pl.pallas_call(
        body,
        out_shape=jax.ShapeDtypeStruct((CHUNK, N), jnp.float32),
        in_specs=[
            pl.BlockSpec(memory_space=pltpu.VMEM),
            pl.BlockSpec(memory_space=pltpu.VMEM),
        ],
        out_specs=pl.BlockSpec(memory_space=pltpu.VMEM),
        scratch_shapes=[
            pltpu.VMEM((M, K_SHARD), jnp.bfloat16),
            pltpu.VMEM((K_SHARD, N), jnp.bfloat16),
            pltpu.VMEM((N_DEV, 2, HALF, N), jnp.int8),
            pltpu.VMEM((N_DEV, 2, HALF, N), jnp.int8),
            pltpu.SemaphoreType.DMA((N_DEV, 2)),
            pltpu.SemaphoreType.DMA((N_DEV, 2)),
        ],
        compiler_params=pltpu.CompilerParams(collective_id=0),
    )(A, B)
